# Initial kernel scaffold; baseline (speedup 1.0000x reference)
import jax
import jax.numpy as jnp
from jax.experimental import pallas as pl


def kernel(inputs, supports, W_ru_0, b_ru_0, W_h_0, b_h_0, W_ru_1, b_ru_1, W_h_1, b_h_1):
    raise NotImplementedError("write your pallas kernel here")


# speedup vs baseline: 1.6227x; 1.6227x over previous
import jax
import jax.numpy as jnp
from jax.experimental import pallas as pl
from jax.experimental.pallas import tpu as pltpu

_T, _B, _N, _I = 12, 16, 512, 2
_H = 64
_L = 2
_S = 2
_K = 3
_NUM_MAT = 1 + _S * (_K - 1)


def _cheb_proj(x, sups, w, b2d):
    c = x.shape[1]
    acc = jnp.dot(x, w[0:c], preferred_element_type=jnp.float32)
    k = 1
    for sm in sups:
        t1 = jnp.dot(sm, x, preferred_element_type=jnp.float32)
        acc = acc + jnp.dot(t1, w[k * c:(k + 1) * c],
                            preferred_element_type=jnp.float32)
        k += 1
        t2 = 2.0 * jnp.dot(sm, t1, preferred_element_type=jnp.float32) - x
        acc = acc + jnp.dot(t2, w[k * c:(k + 1) * c],
                            preferred_element_type=jnp.float32)
        k += 1
    return acc + b2d


def _cell(inp, st, sups, w_ru, b_ru, w_h, b_h):
    x = jnp.concatenate([inp, st], axis=1)
    g = jax.nn.sigmoid(_cheb_proj(x, sups, w_ru, b_ru))
    r = g[:, :_H]
    u = g[:, _H:]
    x2 = jnp.concatenate([inp, r * st], axis=1)
    cand = jnp.tanh(_cheb_proj(x2, sups, w_h, b_h))
    return u * st + (1.0 - u) * cand


def _body(x_ref, sup_ref, wru0_ref, bru0_ref, wh0_ref, bh0_ref,
          wru1_ref, bru1_ref, wh1_ref, bh1_ref, out_ref):
    sups = [sup_ref[0], sup_ref[1]]
    wru0 = wru0_ref[:, :]
    bru0 = bru0_ref[:, :]
    wh0 = wh0_ref[:, :]
    bh0 = bh0_ref[:, :]
    wru1 = wru1_ref[:, :]
    bru1 = bru1_ref[:, :]
    wh1 = wh1_ref[:, :]
    bh1 = bh1_ref[:, :]

    def step(t, carry):
        s0, s1 = carry
        inp = x_ref[t, 0]
        o0 = _cell(inp, s0, sups, wru0, bru0, wh0, bh0)
        o1 = _cell(o0, s1, sups, wru1, bru1, wh1, bh1)
        return (o0, o1)

    z = jnp.zeros((_N, _H), jnp.float32)
    s0, s1 = jax.lax.fori_loop(0, _T, step, (z, z))
    out_ref[0, 0] = s0
    out_ref[1, 0] = s1


def kernel(inputs, supports, W_ru_0, b_ru_0, W_h_0, b_h_0,
           W_ru_1, b_ru_1, W_h_1, b_h_1):
    in0 = (_I + _H) * _NUM_MAT
    in1 = (_H + _H) * _NUM_MAT
    out = pl.pallas_call(
        _body,
        grid=(_B,),
        in_specs=[
            pl.BlockSpec((_T, 1, _N, _I), lambda b: (0, b, 0, 0)),
            pl.BlockSpec((_S, _N, _N), lambda b: (0, 0, 0)),
            pl.BlockSpec((in0, 2 * _H), lambda b: (0, 0)),
            pl.BlockSpec((1, 2 * _H), lambda b: (0, 0)),
            pl.BlockSpec((in0, _H), lambda b: (0, 0)),
            pl.BlockSpec((1, _H), lambda b: (0, 0)),
            pl.BlockSpec((in1, 2 * _H), lambda b: (0, 0)),
            pl.BlockSpec((1, 2 * _H), lambda b: (0, 0)),
            pl.BlockSpec((in1, _H), lambda b: (0, 0)),
            pl.BlockSpec((1, _H), lambda b: (0, 0)),
        ],
        out_specs=pl.BlockSpec((_L, 1, _N, _H), lambda b: (0, b, 0, 0)),
        out_shape=jax.ShapeDtypeStruct((_L, _B, _N, _H), jnp.float32),
        compiler_params=pltpu.CompilerParams(
            dimension_semantics=("arbitrary",)),
    )(inputs, supports,
      W_ru_0, b_ru_0.reshape(1, -1), W_h_0, b_h_0.reshape(1, -1),
      W_ru_1, b_ru_1.reshape(1, -1), W_h_1, b_h_1.reshape(1, -1))
    return out
